# baseline (speedup 1.0000x reference)
import functools

import jax
import jax.numpy as jnp
from jax import lax
from jax.experimental import pallas as pl
from jax.experimental.pallas import tpu as pltpu
from jax.experimental.pallas import tpu_sc as plsc

_B = 16384
_D = 16
_NC = 2
_NS = 16
_NW = _NC * _NS
_BPW = _B // _NW
_CHUNK = 128
_NCH = _BPW // _CHUNK


def _embed_body(user_hbm, item_hbm, uw_hbm, iw_hbm, out_u, out_i,
                idx_u, idx_i, rows_u, rows_i, sem):
    wid = lax.axis_index("s") * _NC + lax.axis_index("c")
    pltpu.sync_copy(user_hbm.at[wid], idx_u)
    pltpu.sync_copy(item_hbm.at[wid], idx_i)
    copies = []
    for j in range(_NCH):
        copies.append(pltpu.async_copy(uw_hbm.at[idx_u.at[j]], rows_u.at[j], sem))
        copies.append(pltpu.async_copy(iw_hbm.at[idx_i.at[j]], rows_i.at[j], sem))
    for c in copies:
        c.wait()
    base = wid * _NCH
    pltpu.sync_copy(rows_u, out_u.at[pl.ds(base, _NCH)])
    pltpu.sync_copy(rows_i, out_i.at[pl.ds(base, _NCH)])


@jax.jit
def kernel(user, item, embed_user_w, embed_item_w):
    call = functools.partial(
        pl.kernel,
        mesh=plsc.VectorSubcoreMesh(core_axis_name="c", subcore_axis_name="s"),
        compiler_params=pltpu.CompilerParams(use_tc_tiling_on_sc=False),
        out_type=(
            jax.ShapeDtypeStruct((_NW * _NCH, _CHUNK, _D), jnp.float32),
            jax.ShapeDtypeStruct((_NW * _NCH, _CHUNK, _D), jnp.float32),
        ),
        scratch_types=[
            pltpu.VMEM((_NCH, _CHUNK), jnp.int32),
            pltpu.VMEM((_NCH, _CHUNK), jnp.int32),
            pltpu.VMEM((_NCH, _CHUNK, _D), jnp.float32),
            pltpu.VMEM((_NCH, _CHUNK, _D), jnp.float32),
            pltpu.SemaphoreType.DMA,
        ],
    )(_embed_body)
    u3 = user.reshape(_NW, _NCH, _CHUNK)
    i3 = item.reshape(_NW, _NCH, _CHUNK)
    out_u, out_i = call(u3, i3, embed_user_w, embed_item_w)
    return out_u.reshape(_B, _D), out_i.reshape(_B, _D)
